# baseline (speedup 1.0000x reference)
import functools

import jax
import jax.numpy as jnp
from jax import lax
from jax.experimental import pallas as pl
from jax.experimental.pallas import tpu as pltpu
from jax.experimental.pallas import tpu_sc as plsc

_N = 10000
_E = 160000
_DI = 32
_DO = 32
_R = 64
_B = 16

_TILES = 32
_CH = 56
_NBUF = 3
_NCH = 90
_EPT = _NCH * _CH
_EPAD = _EPT * _TILES
_NP = 10240
_ROWS_PER_TILE = _NP // 16


def _dense_proj(x, wcat, wself):
    bn = 2000

    def body(x_ref, w1_ref, w2_ref, y_ref, c_ref):
        xb = x_ref[...]
        y = jnp.dot(xb, w1_ref[...], preferred_element_type=jnp.float32)
        y_ref[...] = y.astype(jnp.bfloat16)
        c_ref[...] = jnp.dot(xb, w2_ref[...], preferred_element_type=jnp.float32)

    return pl.pallas_call(
        body,
        grid=(_N // bn,),
        in_specs=[
            pl.BlockSpec((bn, _DI), lambda i: (i, 0)),
            pl.BlockSpec((_DI, _B * _DO), lambda i: (0, 0)),
            pl.BlockSpec((_DI, _DO), lambda i: (0, 0)),
        ],
        out_specs=[
            pl.BlockSpec((bn, _B * _DO), lambda i: (i, 0)),
            pl.BlockSpec((bn, _DO), lambda i: (i, 0)),
        ],
        out_shape=[
            jax.ShapeDtypeStruct((_N, _B * _DO), jnp.bfloat16),
            jax.ShapeDtypeStruct((_N, _DO), jnp.float32),
        ],
    )(x, wcat, wself)


def _combine(a0, a1, curr):
    bn = 2000

    def body(a0_ref, a1_ref, c_ref, o_ref):
        o_ref[...] = jnp.maximum(a0_ref[...] + a1_ref[...] + c_ref[...], 0.0)

    return pl.pallas_call(
        body,
        grid=(_N // bn,),
        in_specs=[pl.BlockSpec((bn, _DO), lambda i: (i, 0))] * 3,
        out_specs=pl.BlockSpec((bn, _DO), lambda i: (i, 0)),
        out_shape=jax.ShapeDtypeStruct((_N, _DO), jnp.float32),
    )(a0, a1, curr)


def _edge_coeffs(et3, nz3, w_comp):
    nb = _EPAD // 1792

    def body(t_ref, n_ref, wc_ref, c_ref):
        th = t_ref[0, 0, :]
        nh = n_ref[0, 0, :]
        onehot = (th[:, None] == lax.broadcasted_iota(jnp.int32, (1792, _R), 1))
        onehot = onehot.astype(jnp.float32) * nh[:, None]
        c_ref[0] = jnp.dot(onehot, wc_ref[...], preferred_element_type=jnp.float32)

    return pl.pallas_call(
        body,
        grid=(nb,),
        in_specs=[
            pl.BlockSpec((1, 1, 1792), lambda i: (i, 0, 0)),
            pl.BlockSpec((1, 1, 1792), lambda i: (i, 0, 0)),
            pl.BlockSpec((_R, _B), lambda i: (0, 0)),
        ],
        out_specs=pl.BlockSpec((1, 1792, _B), lambda i: (i, 0, 0)),
        out_shape=jax.ShapeDtypeStruct((nb, 1792, _B), jnp.float32),
    )(et3, nz3, w_comp).reshape(_EPAD, _B)


def _sc_aggregate(y, src, dst, coeff):
    mesh = plsc.VectorSubcoreMesh(core_axis_name="c", subcore_axis_name="s")

    @functools.partial(
        pl.kernel,
        out_type=jax.ShapeDtypeStruct((2 * _NP, _DO), jnp.float32),
        mesh=mesh,
        compiler_params=pltpu.CompilerParams(use_tc_tiling_on_sc=False),
        scratch_types=[
            pltpu.VMEM((_EPT,), jnp.int32),
            pltpu.VMEM((_NCH, _CH), jnp.int32),
            pltpu.VMEM((_NBUF, _CH, _B), jnp.float32),
            pltpu.VMEM((_NBUF, _CH, 4, 128), jnp.bfloat16),
            pltpu.VMEM((_CH, _DO), jnp.float32),
            pltpu.VMEM_SHARED((_NP, _DO), jnp.float32),
            pltpu.SemaphoreType.DMA,
            pltpu.SemaphoreType.DMA,
            pltpu.SemaphoreType.DMA,
            pltpu.SemaphoreType.DMA,
            pltpu.SemaphoreType.DMA,
        ],
    )
    def k(y_hbm, src_hbm, dst_hbm, c_hbm, out_hbm,
          src_v, dst_v, c_v, y_v, msg_v, agg_sh, sem0, sem1, sem2, sem3, sem4):
        core = lax.axis_index("c")
        sub = lax.axis_index("s")
        wid = core * 16 + sub
        base = wid * _EPT
        sems = (sem0, sem1, sem2, sem3, sem4)

        pltpu.sync_copy(src_hbm.at[pl.ds(base, _EPT)], src_v)
        pltpu.sync_copy(dst_hbm.at[wid], dst_v)

        z16 = jnp.zeros((16,), jnp.float32)

        def zrow(i, c0):
            msg_v[i, pl.ds(0, 16)] = z16
            msg_v[i, pl.ds(16, 16)] = z16
            return c0

        lax.fori_loop(0, _CH, zrow, 0)
        st = sub * _ROWS_PER_TILE
        for r in range(_ROWS_PER_TILE // 40):
            pltpu.sync_copy(msg_v.at[pl.ds(0, 40)],
                            agg_sh.at[pl.ds(st + r * 40, 40)])
        plsc.subcore_barrier()

        def issue(g, slot):
            pltpu.async_copy(
                c_hbm.at[pl.ds(base + g * _CH, _CH), :], c_v.at[slot], sems[slot])
            pltpu.async_copy(
                y_hbm.at[src_v.at[pl.ds(g * _CH, _CH)]], y_v.at[slot], sems[slot])

        def drain(slot):
            pltpu.make_async_copy(
                c_hbm.at[pl.ds(0, _CH), :], c_v.at[slot], sems[slot]).wait()
            pltpu.make_async_copy(
                y_hbm.at[src_v.at[pl.ds(0, _CH)]], y_v.at[slot], sems[slot]).wait()

        for s in range(_NBUF):
            issue(s, s)

        def ring(gp, carry):
            for slot in range(_NBUF):
                g = _NBUF * gp + slot
                drain(slot)
                yb = y_v.at[slot]
                cb_ref = c_v.at[slot]

                def edge2(jj, c2):
                    for j in (2 * jj, 2 * jj + 1):
                        cvec = cb_ref[j, :]
                        z = jnp.zeros((16,), jnp.float32)
                        lo = [z, z, z, z]
                        hi = [z, z, z, z]
                        for b in range(_B):
                            cb = cvec[b]
                            s, off = b // 4, (b % 4) * _DO
                            ylo = yb[j, s, pl.ds(off, 16)].astype(jnp.float32)
                            yhi = yb[j, s, pl.ds(off + 16, 16)].astype(jnp.float32)
                            lo[b % 4] = lo[b % 4] + cb * ylo
                            hi[b % 4] = hi[b % 4] + cb * yhi
                        msg_v[j, pl.ds(0, 16)] = (lo[0] + lo[1]) + (lo[2] + lo[3])
                        msg_v[j, pl.ds(16, 16)] = (hi[0] + hi[1]) + (hi[2] + hi[3])
                    return c2

                lax.fori_loop(0, _CH // 2, edge2, 0)
                pltpu.sync_copy(msg_v, agg_sh.at[dst_v.at[g]], add=True)

                @pl.when(g + _NBUF < _NCH)
                def _():
                    issue(g + _NBUF, slot)
            return carry

        lax.fori_loop(0, _NCH // _NBUF, ring, 0)
        plsc.subcore_barrier()
        pltpu.sync_copy(agg_sh.at[pl.ds(st, _ROWS_PER_TILE)],
                        out_hbm.at[pl.ds(core * _NP + st, _ROWS_PER_TILE)])

    return k(y, src, dst, coeff)


def kernel(x, edge_index, edge_type, norm, weight, w_comp, self_loop_weight):
    wcat = weight.transpose(1, 0, 2).reshape(_DI, _B * _DO)
    y, curr = _dense_proj(x, wcat, self_loop_weight)
    y = y.reshape(_N, 4, 128)

    pad = _EPAD - _E
    src = jnp.pad(edge_index[0], (0, pad))
    dst = jnp.pad(edge_index[1], (0, pad)).reshape(_TILES, _NCH, _CH)
    et3 = jnp.pad(edge_type, (0, pad)).reshape(_EPAD // 1792, 1, 1792)
    nz3 = jnp.pad(norm, (0, pad)).reshape(_EPAD // 1792, 1, 1792)
    coeff = _edge_coeffs(et3, nz3, w_comp)

    agg = _sc_aggregate(y, src, dst, coeff).reshape(2, _NP, _DO)[:, :_N, :]
    return _combine(agg[0], agg[1], curr)


# speedup vs baseline: 1.0904x; 1.0904x over previous
import functools

import jax
import jax.numpy as jnp
from jax import lax
from jax.experimental import pallas as pl
from jax.experimental.pallas import tpu as pltpu
from jax.experimental.pallas import tpu_sc as plsc

_N = 10000
_E = 160000
_DI = 32
_DO = 32
_R = 64
_B = 16

_TILES = 32
_CH = 56
_NBUF = 3
_NCH = 90
_EPT = _NCH * _CH
_EPAD = _EPT * _TILES
_NP = 10240
_ROWS_PER_TILE = _NP // 16


def _dense_proj(x, wcat, wself, et3, nz3, w_comp):
    bn = 2000
    eb = _EPAD // 5

    def body(x_ref, w1_ref, w2_ref, t_ref, n_ref, wc_ref,
             y_ref, c_ref, co_ref):
        xb = x_ref[...]
        y = jnp.dot(xb, w1_ref[...], preferred_element_type=jnp.float32)
        y_ref[...] = y.astype(jnp.bfloat16)
        c_ref[...] = jnp.dot(xb, w2_ref[...], preferred_element_type=jnp.float32)
        th = t_ref[0, 0, :]
        nh = n_ref[0, 0, :]
        onehot = (th[:, None] == lax.broadcasted_iota(jnp.int32, (eb, _R), 1))
        onehot = onehot.astype(jnp.float32) * nh[:, None]
        co_ref[0] = jnp.dot(onehot, wc_ref[...], preferred_element_type=jnp.float32)

    return pl.pallas_call(
        body,
        grid=(_N // bn,),
        in_specs=[
            pl.BlockSpec((bn, _DI), lambda i: (i, 0)),
            pl.BlockSpec((_DI, _B * _DO), lambda i: (0, 0)),
            pl.BlockSpec((_DI, _DO), lambda i: (0, 0)),
            pl.BlockSpec((1, 1, eb), lambda i: (i, 0, 0)),
            pl.BlockSpec((1, 1, eb), lambda i: (i, 0, 0)),
            pl.BlockSpec((_R, _B), lambda i: (0, 0)),
        ],
        out_specs=[
            pl.BlockSpec((bn, _B * _DO), lambda i: (i, 0)),
            pl.BlockSpec((bn, _DO), lambda i: (i, 0)),
            pl.BlockSpec((1, eb, _B), lambda i: (i, 0, 0)),
        ],
        out_shape=[
            jax.ShapeDtypeStruct((_N, _B * _DO), jnp.bfloat16),
            jax.ShapeDtypeStruct((_N, _DO), jnp.float32),
            jax.ShapeDtypeStruct((5, eb, _B), jnp.float32),
        ],
    )(x, wcat, wself, et3, nz3, w_comp)


def _combine(a0, a1, curr):
    bn = 2000

    def body(a0_ref, a1_ref, c_ref, o_ref):
        o_ref[...] = jnp.maximum(a0_ref[...] + a1_ref[...] + c_ref[...], 0.0)

    return pl.pallas_call(
        body,
        grid=(_N // bn,),
        in_specs=[pl.BlockSpec((bn, _DO), lambda i: (i, 0))] * 3,
        out_specs=pl.BlockSpec((bn, _DO), lambda i: (i, 0)),
        out_shape=jax.ShapeDtypeStruct((_N, _DO), jnp.float32),
    )(a0, a1, curr)


def _sc_aggregate(y, src, dst, coeff):
    mesh = plsc.VectorSubcoreMesh(core_axis_name="c", subcore_axis_name="s")

    @functools.partial(
        pl.kernel,
        out_type=jax.ShapeDtypeStruct((2 * _NP, _DO), jnp.float32),
        mesh=mesh,
        compiler_params=pltpu.CompilerParams(use_tc_tiling_on_sc=False),
        scratch_types=[
            pltpu.VMEM((_EPT,), jnp.int32),
            pltpu.VMEM((_NCH, _CH), jnp.int32),
            pltpu.VMEM((_NBUF, _CH, _B), jnp.float32),
            pltpu.VMEM((_NBUF, _CH, 4, 128), jnp.bfloat16),
            pltpu.VMEM((_CH, _DO), jnp.float32),
            pltpu.VMEM_SHARED((_NP, _DO), jnp.float32),
            pltpu.SemaphoreType.DMA,
            pltpu.SemaphoreType.DMA,
            pltpu.SemaphoreType.DMA,
            pltpu.SemaphoreType.DMA,
            pltpu.SemaphoreType.DMA,
        ],
    )
    def k(y_hbm, src_hbm, dst_hbm, c_hbm, out_hbm,
          src_v, dst_v, c_v, y_v, msg_v, agg_sh, sem0, sem1, sem2, sem3, sem4):
        core = lax.axis_index("c")
        sub = lax.axis_index("s")
        wid = core * 16 + sub
        base = wid * _EPT
        sems = (sem0, sem1, sem2, sem3, sem4)

        pltpu.sync_copy(src_hbm.at[pl.ds(base, _EPT)], src_v)
        pltpu.sync_copy(dst_hbm.at[wid], dst_v)

        z16 = jnp.zeros((16,), jnp.float32)

        def zrow(i, c0):
            msg_v[i, pl.ds(0, 16)] = z16
            msg_v[i, pl.ds(16, 16)] = z16
            return c0

        lax.fori_loop(0, _CH, zrow, 0)
        st = sub * _ROWS_PER_TILE
        for r in range(_ROWS_PER_TILE // 40):
            pltpu.sync_copy(msg_v.at[pl.ds(0, 40)],
                            agg_sh.at[pl.ds(st + r * 40, 40)])
        plsc.subcore_barrier()

        def issue(g, slot):
            pltpu.async_copy(
                c_hbm.at[pl.ds(base + g * _CH, _CH), :], c_v.at[slot], sems[slot])
            pltpu.async_copy(
                y_hbm.at[src_v.at[pl.ds(g * _CH, _CH)]], y_v.at[slot], sems[slot])

        def drain(slot):
            pltpu.make_async_copy(
                c_hbm.at[pl.ds(0, _CH), :], c_v.at[slot], sems[slot]).wait()
            pltpu.make_async_copy(
                y_hbm.at[src_v.at[pl.ds(0, _CH)]], y_v.at[slot], sems[slot]).wait()

        for s in range(_NBUF):
            issue(s, s)

        def ring(gp, carry):
            for slot in range(_NBUF):
                g = _NBUF * gp + slot
                drain(slot)
                yb = y_v.at[slot]
                cb_ref = c_v.at[slot]

                def edge2(jj, c2):
                    for j in (2 * jj, 2 * jj + 1):
                        cvec = cb_ref[j, :]
                        z = jnp.zeros((16,), jnp.float32)
                        lo = [z, z, z, z]
                        hi = [z, z, z, z]
                        for b in range(_B):
                            cb = cvec[b]
                            s, off = b // 4, (b % 4) * _DO
                            ylo = yb[j, s, pl.ds(off, 16)].astype(jnp.float32)
                            yhi = yb[j, s, pl.ds(off + 16, 16)].astype(jnp.float32)
                            lo[b % 4] = lo[b % 4] + cb * ylo
                            hi[b % 4] = hi[b % 4] + cb * yhi
                        msg_v[j, pl.ds(0, 16)] = (lo[0] + lo[1]) + (lo[2] + lo[3])
                        msg_v[j, pl.ds(16, 16)] = (hi[0] + hi[1]) + (hi[2] + hi[3])
                    return c2

                lax.fori_loop(0, _CH // 2, edge2, 0)
                pltpu.sync_copy(msg_v, agg_sh.at[dst_v.at[g]], add=True)

                @pl.when(g + _NBUF < _NCH)
                def _():
                    issue(g + _NBUF, slot)
            return carry

        lax.fori_loop(0, _NCH // _NBUF, ring, 0)
        plsc.subcore_barrier()
        pltpu.sync_copy(agg_sh.at[pl.ds(st, _ROWS_PER_TILE)],
                        out_hbm.at[pl.ds(core * _NP + st, _ROWS_PER_TILE)])

    return k(y, src, dst, coeff)


def kernel(x, edge_index, edge_type, norm, weight, w_comp, self_loop_weight):
    wcat = weight.transpose(1, 0, 2).reshape(_DI, _B * _DO)

    pad = _EPAD - _E
    eb = _EPAD // 5
    src = jnp.pad(edge_index[0], (0, pad))
    dst = jnp.pad(edge_index[1], (0, pad)).reshape(_TILES, _NCH, _CH)
    et3 = jnp.pad(edge_type, (0, pad)).reshape(5, 1, eb)
    nz3 = jnp.pad(norm, (0, pad)).reshape(5, 1, eb)

    y, curr, coeff = _dense_proj(x, wcat, self_loop_weight, et3, nz3, w_comp)
    y = y.reshape(_N, 4, 128)
    coeff = coeff.reshape(_EPAD, _B)

    agg = _sc_aggregate(y, src, dst, coeff).reshape(2, _NP, _DO)[:, :_N, :]
    return _combine(agg[0], agg[1], curr)


# speedup vs baseline: 1.0927x; 1.0021x over previous
import functools

import jax
import jax.numpy as jnp
from jax import lax
from jax.experimental import pallas as pl
from jax.experimental.pallas import tpu as pltpu
from jax.experimental.pallas import tpu_sc as plsc

_N = 10000
_E = 160000
_DI = 32
_DO = 32
_R = 64
_B = 16

_TILES = 32
_CH = 56
_NBUF = 3
_NCH = 90
_EPT = _NCH * _CH
_EPAD = _EPT * _TILES
_NP = 10240
_ROWS_PER_TILE = _NP // 16


def _dense_proj(x, wcat, wself, et3, nz3, w_comp):
    bn = 2000
    eb = _EPAD // 5

    def body(x_ref, w1_ref, w2_ref, t_ref, n_ref, wc_ref,
             y_ref, c_ref, co_ref):
        xb = x_ref[...]
        y = jnp.dot(xb, w1_ref[...], preferred_element_type=jnp.float32)
        y_ref[...] = y.astype(jnp.bfloat16)
        c_ref[...] = jnp.dot(xb, w2_ref[...], preferred_element_type=jnp.float32)
        th = t_ref[0, 0, :]
        nh = n_ref[0, 0, :]
        onehot = (th[:, None] == lax.broadcasted_iota(jnp.int32, (eb, _R), 1))
        onehot = onehot.astype(jnp.float32) * nh[:, None]
        co_ref[0] = jnp.dot(onehot, wc_ref[...], preferred_element_type=jnp.float32)

    return pl.pallas_call(
        body,
        grid=(_N // bn,),
        in_specs=[
            pl.BlockSpec((bn, _DI), lambda i: (i, 0)),
            pl.BlockSpec((_DI, _B * _DO), lambda i: (0, 0)),
            pl.BlockSpec((_DI, _DO), lambda i: (0, 0)),
            pl.BlockSpec((1, 1, eb), lambda i: (i, 0, 0)),
            pl.BlockSpec((1, 1, eb), lambda i: (i, 0, 0)),
            pl.BlockSpec((_R, _B), lambda i: (0, 0)),
        ],
        out_specs=[
            pl.BlockSpec((bn, _B * _DO), lambda i: (i, 0)),
            pl.BlockSpec((bn, _DO), lambda i: (i, 0)),
            pl.BlockSpec((1, eb, _B), lambda i: (i, 0, 0)),
        ],
        out_shape=[
            jax.ShapeDtypeStruct((_N, _B * _DO), jnp.bfloat16),
            jax.ShapeDtypeStruct((_N, _DO), jnp.float32),
            jax.ShapeDtypeStruct((5, eb, _B), jnp.float32),
        ],
    )(x, wcat, wself, et3, nz3, w_comp)


def _combine(a0, a1, curr):
    bn = 2000

    def body(a0_ref, a1_ref, c_ref, o_ref):
        o_ref[...] = jnp.maximum(a0_ref[...] + a1_ref[...] + c_ref[...], 0.0)

    return pl.pallas_call(
        body,
        grid=(_N // bn,),
        in_specs=[pl.BlockSpec((bn, _DO), lambda i: (i, 0))] * 3,
        out_specs=pl.BlockSpec((bn, _DO), lambda i: (i, 0)),
        out_shape=jax.ShapeDtypeStruct((_N, _DO), jnp.float32),
    )(a0, a1, curr)


def _sc_aggregate(y, src, dst, coeff):
    mesh = plsc.VectorSubcoreMesh(core_axis_name="c", subcore_axis_name="s")

    @functools.partial(
        pl.kernel,
        out_type=jax.ShapeDtypeStruct((2 * _NP, _DO), jnp.float32),
        mesh=mesh,
        compiler_params=pltpu.CompilerParams(use_tc_tiling_on_sc=False),
        scratch_types=[
            pltpu.VMEM((_EPT,), jnp.int32),
            pltpu.VMEM((_NCH, _CH), jnp.int32),
            pltpu.VMEM((_NBUF, _CH, _B), jnp.float32),
            pltpu.VMEM((_NBUF, _CH, 4, 128), jnp.bfloat16),
            pltpu.VMEM((_CH, _DO), jnp.float32),
            pltpu.VMEM_SHARED((_NP, _DO), jnp.float32),
            pltpu.SemaphoreType.DMA,
            pltpu.SemaphoreType.DMA,
            pltpu.SemaphoreType.DMA,
            pltpu.SemaphoreType.DMA,
            pltpu.SemaphoreType.DMA,
        ],
    )
    def k(y_hbm, src_hbm, dst_hbm, c_hbm, out_hbm,
          src_v, dst_v, c_v, y_v, msg_v, agg_sh, sem0, sem1, sem2, sem3, sem4):
        core = lax.axis_index("c")
        sub = lax.axis_index("s")
        wid = core * 16 + sub
        base = wid * _EPT
        sems = (sem0, sem1, sem2, sem3, sem4)

        pltpu.sync_copy(src_hbm.at[pl.ds(base, _EPT)], src_v)
        pltpu.sync_copy(dst_hbm.at[wid], dst_v)

        z16 = jnp.zeros((16,), jnp.float32)

        def zrow(i, c0):
            msg_v[i, pl.ds(0, 16)] = z16
            msg_v[i, pl.ds(16, 16)] = z16
            return c0

        lax.fori_loop(0, _CH, zrow, 0)
        st = sub * _ROWS_PER_TILE
        nz = _ROWS_PER_TILE // 40
        for r in range(nz):
            pltpu.async_copy(msg_v.at[pl.ds(0, 40)],
                             agg_sh.at[pl.ds(st + r * 40, 40)], sem4)
        for r in range(nz):
            pltpu.make_async_copy(msg_v.at[pl.ds(0, 40)],
                                  agg_sh.at[pl.ds(st, 40)], sem4).wait()
        plsc.subcore_barrier()

        def issue(g, slot):
            pltpu.async_copy(
                c_hbm.at[pl.ds(base + g * _CH, _CH), :], c_v.at[slot], sems[slot])
            pltpu.async_copy(
                y_hbm.at[src_v.at[pl.ds(g * _CH, _CH)]], y_v.at[slot], sems[slot])

        def drain(slot):
            pltpu.make_async_copy(
                c_hbm.at[pl.ds(0, _CH), :], c_v.at[slot], sems[slot]).wait()
            pltpu.make_async_copy(
                y_hbm.at[src_v.at[pl.ds(0, _CH)]], y_v.at[slot], sems[slot]).wait()

        for s in range(_NBUF):
            issue(s, s)

        def ring(gp, carry):
            for slot in range(_NBUF):
                g = _NBUF * gp + slot
                drain(slot)
                yb = y_v.at[slot]
                cb_ref = c_v.at[slot]

                def edge2(jj, c2):
                    for j in (2 * jj, 2 * jj + 1):
                        cvec = cb_ref[j, :]
                        z = jnp.zeros((16,), jnp.float32)
                        lo = [z, z, z, z]
                        hi = [z, z, z, z]
                        for b in range(_B):
                            cb = cvec[b]
                            s, off = b // 4, (b % 4) * _DO
                            ylo = yb[j, s, pl.ds(off, 16)].astype(jnp.float32)
                            yhi = yb[j, s, pl.ds(off + 16, 16)].astype(jnp.float32)
                            lo[b % 4] = lo[b % 4] + cb * ylo
                            hi[b % 4] = hi[b % 4] + cb * yhi
                        msg_v[j, pl.ds(0, 16)] = (lo[0] + lo[1]) + (lo[2] + lo[3])
                        msg_v[j, pl.ds(16, 16)] = (hi[0] + hi[1]) + (hi[2] + hi[3])
                    return c2

                lax.fori_loop(0, _CH // 2, edge2, 0)
                pltpu.sync_copy(msg_v, agg_sh.at[dst_v.at[g]], add=True)

                @pl.when(g + _NBUF < _NCH)
                def _():
                    issue(g + _NBUF, slot)
            return carry

        lax.fori_loop(0, _NCH // _NBUF, ring, 0)
        plsc.subcore_barrier()
        pltpu.sync_copy(agg_sh.at[pl.ds(st, _ROWS_PER_TILE)],
                        out_hbm.at[pl.ds(core * _NP + st, _ROWS_PER_TILE)])

    return k(y, src, dst, coeff)


def kernel(x, edge_index, edge_type, norm, weight, w_comp, self_loop_weight):
    wcat = weight.transpose(1, 0, 2).reshape(_DI, _B * _DO)

    pad = _EPAD - _E
    eb = _EPAD // 5
    src = jnp.pad(edge_index[0], (0, pad))
    dst = jnp.pad(edge_index[1], (0, pad)).reshape(_TILES, _NCH, _CH)
    et3 = jnp.pad(edge_type, (0, pad)).reshape(5, 1, eb)
    nz3 = jnp.pad(norm, (0, pad)).reshape(5, 1, eb)

    y, curr, coeff = _dense_proj(x, wcat, self_loop_weight, et3, nz3, w_comp)
    y = y.reshape(_N, 4, 128)
    coeff = coeff.reshape(_EPAD, _B)

    agg = _sc_aggregate(y, src, dst, coeff).reshape(2, _NP, _DO)[:, :_N, :]
    return _combine(agg[0], agg[1], curr)
